# Initial kernel scaffold; baseline (speedup 1.0000x reference)
import jax
import jax.numpy as jnp
from jax.experimental import pallas as pl


def kernel(hidden_states, gate_w, shared_gate_w, shared_up_w, shared_down_w, routed_gate_w, routed_up_w, routed_down_w):
    raise NotImplementedError("write your pallas kernel here")


# speedup vs baseline: 8.5703x; 8.5703x over previous
import functools

import jax
import jax.numpy as jnp
from jax import lax
from jax.experimental import pallas as pl
from jax.experimental.pallas import tpu as pltpu
from jax.experimental.pallas import tpu_sc as plsc

B, S, H, I, E = 2, 2048, 1024, 512, 64
N = B * S
BT = 128
NT = N // BT
VMAX = 96
VPAD = 128


def _silu(x):
    return x * jax.nn.sigmoid(x)


def _router_body(x_ref, gw_ref, sg_ref, su_ref, sd_ref,
                 xsh_ref, pos_ref, meta_ref,
                 oh_s, rank_s, cnt_s):
    t = pl.program_id(0)

    @pl.when(t == 0)
    def _():
        cnt_s[...] = jnp.zeros_like(cnt_s)

    x = x_ref[...]

    sh = _silu(x @ sg_ref[...]) * (x @ su_ref[...])
    sh = sh @ sd_ref[...]

    logits = x @ gw_ref[...]
    m = jnp.max(logits, axis=1, keepdims=True)
    eiota = lax.broadcasted_iota(jnp.int32, (BT, E), 1)
    ids = jnp.min(jnp.where(logits == m, eiota, E), axis=1, keepdims=True)
    xw = x * jax.nn.sigmoid(m)

    xsh_ref[:, :H] = xw
    xsh_ref[:, H:] = sh

    onehot = (ids == eiota).astype(jnp.float32)
    ri = lax.broadcasted_iota(jnp.int32, (BT, BT), 0)
    ci = lax.broadcasted_iota(jnp.int32, (BT, BT), 1)
    lt = (ci < ri).astype(jnp.float32)
    prefix = lt @ onehot
    rc = cnt_s[...]
    rank = jnp.sum(onehot * (prefix + rc), axis=1, keepdims=True)
    cnt_s[...] = rc + jnp.sum(onehot, axis=0, keepdims=True)

    oh_s[pl.ds(t * BT, BT), :] = onehot
    rank_s[pl.ds(t * BT, BT), :] = rank

    @pl.when(t == NT - 1)
    def _epilogue():
        oh = oh_s[...]
        ones_n = jnp.ones((N, 1), jnp.float32)
        cnt_col = lax.dot_general(oh, ones_n,
                                  (((0,), (0,)), ((), ())))
        cnt_row = cnt_s[...]
        mr = lax.broadcasted_iota(jnp.int32, (E, E), 0)
        mc = lax.broadcasted_iota(jnp.int32, (E, E), 1)
        ut = (mr < mc).astype(jnp.float32)
        off_row = cnt_row @ ut

        pos = rank_s[...] + jnp.sum(
            oh * jnp.broadcast_to(off_row, (N, E)), axis=1, keepdims=True)
        pos_ref[...] = pos.astype(jnp.int32)

        base = lax.broadcasted_iota(jnp.int32, (NT, 1), 0) * BT
        offb = jnp.broadcast_to(off_row, (NT, E))
        elo = jnp.sum((offb <= base).astype(jnp.int32), axis=1,
                      keepdims=True) - 1
        ehi = jnp.sum((offb <= base + (BT - 1)).astype(jnp.int32), axis=1,
                      keepdims=True) - 1
        nv = (ehi - elo + 1).astype(jnp.float32)
        tr = lax.broadcasted_iota(jnp.int32, (NT, NT), 0)
        tc = lax.broadcasted_iota(jnp.int32, (NT, NT), 1)
        lt_t = (tc < tr).astype(jnp.float32)
        vstart = lt_t @ nv
        vtotal = jnp.sum(nv)

        vi = lax.broadcasted_iota(jnp.int32, (VPAD, 1), 0)
        vstart_row = lax.dot_general(
            nv, lt_t, (((0,), (1,)), ((), ())))
        vsb = jnp.broadcast_to(vstart_row, (VPAD, NT))
        vif = vi.astype(jnp.float32)
        tv = jnp.sum((vsb <= vif).astype(jnp.int32), axis=1,
                     keepdims=True) - 1
        tiota = lax.broadcasted_iota(jnp.int32, (VPAD, NT), 1)
        oh_tv = (tv == tiota).astype(jnp.float32)
        elo_v = oh_tv @ elo.astype(jnp.float32)
        ehi_v = oh_tv @ ehi.astype(jnp.float32)
        vstart_v = oh_tv @ vstart
        ev = elo_v + vif - vstart_v
        ev = jnp.minimum(ev, ehi_v)
        eviota = lax.broadcasted_iota(jnp.int32, (VPAD, E), 1)
        oh_ev = (ev.astype(jnp.int32) == eviota).astype(jnp.float32)
        seg_start = jnp.sum(oh_ev * jnp.broadcast_to(off_row, (VPAD, E)),
                            axis=1, keepdims=True)
        seg_cnt = oh_ev @ cnt_col
        tbase = tv.astype(jnp.float32) * BT
        valid = vif < vtotal
        rs = jnp.clip(seg_start - tbase, 0.0, float(BT))
        re = jnp.clip(seg_start + seg_cnt - tbase, 0.0, float(BT))
        rs = jnp.where(valid, rs, 0.0)
        re = jnp.where(valid, re, 0.0)

        meta = jnp.concatenate(
            [tv, ev.astype(jnp.int32), rs.astype(jnp.int32),
             re.astype(jnp.int32)], axis=1)
        meta_ref[...] = meta


def _router_shared(flat, gate_w, sg, su, sd):
    return pl.pallas_call(
        _router_body,
        grid=(NT,),
        in_specs=[
            pl.BlockSpec((BT, H), lambda t: (t, 0)),
            pl.BlockSpec((H, E), lambda t: (0, 0)),
            pl.BlockSpec((H, I), lambda t: (0, 0)),
            pl.BlockSpec((H, I), lambda t: (0, 0)),
            pl.BlockSpec((I, H), lambda t: (0, 0)),
        ],
        out_specs=[
            pl.BlockSpec((BT, 2 * H), lambda t: (t, 0)),
            pl.BlockSpec((N, 1), lambda t: (0, 0)),
            pl.BlockSpec((VPAD, 4), lambda t: (0, 0)),
        ],
        out_shape=[
            jax.ShapeDtypeStruct((N, 2 * H), jnp.float32),
            jax.ShapeDtypeStruct((N, 1), jnp.int32),
            jax.ShapeDtypeStruct((VPAD, 4), jnp.int32),
        ],
        scratch_shapes=[
            pltpu.VMEM((N, E), jnp.float32),
            pltpu.VMEM((N, 1), jnp.float32),
            pltpu.VMEM((1, E), jnp.float32),
        ],
    )(flat, gate_w, sg, su, sd)


def _grouped_body(meta_ref, xsh_ref, wg_ref, wu_ref, wd_ref, out_ref):
    v = pl.program_id(0)
    vt = meta_ref[v, 0]
    vt_prev = meta_ref[jnp.maximum(v - 1, 0), 0]
    first = jnp.logical_or(v == 0, vt != vt_prev)
    rs = meta_ref[v, 2]
    re = meta_ref[v, 3]

    blk = xsh_ref[...]
    xs = blk[:, :H]
    ri = lax.broadcasted_iota(jnp.int32, (BT, 1), 0)
    msk = jnp.logical_and(ri >= rs, ri < re)
    xm = jnp.where(msk, xs, 0.0)
    g = xm @ wg_ref[0]
    u = xm @ wu_ref[0]
    y = (_silu(g) * u) @ wd_ref[0]

    @pl.when(first)
    def _():
        out_ref[...] = blk[:, H:] + y

    @pl.when(jnp.logical_not(first))
    def _():
        out_ref[...] = out_ref[...] + y


def _grouped(meta, xsh, wg, wu, wd):
    grid_spec = pltpu.PrefetchScalarGridSpec(
        num_scalar_prefetch=1,
        grid=(VMAX,),
        in_specs=[
            pl.BlockSpec((BT, 2 * H), lambda v, m: (m[v, 0], 0)),
            pl.BlockSpec((1, H, I), lambda v, m: (m[v, 1], 0, 0)),
            pl.BlockSpec((1, H, I), lambda v, m: (m[v, 1], 0, 0)),
            pl.BlockSpec((1, I, H), lambda v, m: (m[v, 1], 0, 0)),
        ],
        out_specs=pl.BlockSpec((BT, H), lambda v, m: (m[v, 0], 0)),
    )
    return pl.pallas_call(
        _grouped_body,
        grid_spec=grid_spec,
        out_shape=jax.ShapeDtypeStruct((N, H), jnp.float32),
    )(meta, xsh, wg, wu, wd)


def _sc_mesh():
    return plsc.VectorSubcoreMesh(core_axis_name="c", subcore_axis_name="s")


_NW = 32
_RPW = N // _NW


def _scatter_rows(src, pos):
    chunk = 32

    @functools.partial(
        pl.kernel,
        out_type=jax.ShapeDtypeStruct((N, 2 * H), jnp.float32),
        mesh=_sc_mesh(),
        scratch_types=[
            pltpu.VMEM((chunk,), jnp.int32),
            pltpu.VMEM((chunk, 2 * H), jnp.float32),
            pltpu.SemaphoreType.DMA,
        ],
    )
    def k(src_hbm, pos_hbm, out_hbm, idx_v, rows_v, sem):
        wid = lax.axis_index("s") * 2 + lax.axis_index("c")
        base = wid * _RPW
        for ch in range(_RPW // chunk):
            off = base + ch * chunk
            pltpu.sync_copy(pos_hbm.at[pl.ds(off, chunk)], idx_v)
            pltpu.sync_copy(src_hbm.at[pl.ds(off, chunk)], rows_v)
            pltpu.async_copy(rows_v, out_hbm.at[idx_v], sem).wait()

    return k(src, pos)


def _gather_rows(ys, pos):
    chunk = 64

    @functools.partial(
        pl.kernel,
        out_type=jax.ShapeDtypeStruct((N, H), jnp.float32),
        mesh=_sc_mesh(),
        scratch_types=[
            pltpu.VMEM((chunk,), jnp.int32),
            pltpu.VMEM((chunk, H), jnp.float32),
            pltpu.SemaphoreType.DMA,
        ],
    )
    def k(ys_hbm, pos_hbm, out_hbm, idx_v, rows_v, sem):
        wid = lax.axis_index("s") * 2 + lax.axis_index("c")
        base = wid * _RPW
        for ch in range(_RPW // chunk):
            off = base + ch * chunk
            pltpu.sync_copy(pos_hbm.at[pl.ds(off, chunk)], idx_v)
            pltpu.async_copy(ys_hbm.at[idx_v], rows_v, sem).wait()
            pltpu.sync_copy(rows_v, out_hbm.at[pl.ds(off, chunk)])

    return k(ys, pos)


def kernel(hidden_states, gate_w, shared_gate_w, shared_up_w, shared_down_w,
           routed_gate_w, routed_up_w, routed_down_w):
    flat = hidden_states.reshape(N, H)
    xsh, pos2d, meta = _router_shared(flat, gate_w, shared_gate_w,
                                      shared_up_w, shared_down_w)
    pos = pos2d.reshape(N)
    xsh_sorted = _scatter_rows(xsh, pos)
    ys = _grouped(meta, xsh_sorted, routed_gate_w, routed_up_w, routed_down_w)
    out = _gather_rows(ys, pos)
    return out.reshape(B, S, H)


# speedup vs baseline: 9.2231x; 1.0762x over previous
import functools

import jax
import jax.numpy as jnp
from jax import lax
from jax.experimental import pallas as pl
from jax.experimental.pallas import tpu as pltpu
from jax.experimental.pallas import tpu_sc as plsc

B, S, H, I, E = 2, 2048, 1024, 512, 64
N = B * S
BT = 512
NT = N // BT
VMAX = 72
VPAD = 128
SCW = 128


def _silu(x):
    return x * jax.nn.sigmoid(x)


def _router_body(x_ref, gw_ref, score_ref, pos_ref, meta_ref,
                 oh_s, rank_s, cnt_s):
    t = pl.program_id(0)

    @pl.when(t == 0)
    def _():
        cnt_s[...] = jnp.zeros_like(cnt_s)

    x = x_ref[...]

    logits = x @ gw_ref[...]
    m = jnp.max(logits, axis=1, keepdims=True)
    eiota = lax.broadcasted_iota(jnp.int32, (BT, E), 1)
    ids = jnp.min(jnp.where(logits == m, eiota, E), axis=1, keepdims=True)
    score_ref[...] = jnp.broadcast_to(jax.nn.sigmoid(m), (BT, SCW))

    onehot = (ids == eiota).astype(jnp.float32)
    ri = lax.broadcasted_iota(jnp.int32, (BT, BT), 0)
    ci = lax.broadcasted_iota(jnp.int32, (BT, BT), 1)
    lt = (ci < ri).astype(jnp.float32)
    prefix = lt @ onehot
    rc = cnt_s[...]
    rank = jnp.sum(onehot * (prefix + rc), axis=1, keepdims=True)
    cnt_s[...] = rc + jnp.sum(onehot, axis=0, keepdims=True)

    oh_s[pl.ds(t * BT, BT), :] = onehot
    rank_s[pl.ds(t * BT, BT), :] = rank

    @pl.when(t == NT - 1)
    def _epilogue():
        oh = oh_s[...]
        cnt_row = cnt_s[...]
        c_hi = jnp.floor(cnt_row * (1.0 / 256.0))
        c_lo = cnt_row - 256.0 * c_hi
        mr = lax.broadcasted_iota(jnp.int32, (E, E), 0)
        mc = lax.broadcasted_iota(jnp.int32, (E, E), 1)
        ut = (mr < mc).astype(jnp.float32)
        off_row = (c_hi @ ut) * 256.0 + c_lo @ ut

        pos = rank_s[...] + jnp.sum(
            oh * jnp.broadcast_to(off_row, (N, E)), axis=1, keepdims=True)
        pos_ref[...] = pos.astype(jnp.int32)

        base = lax.broadcasted_iota(jnp.int32, (NT, 1), 0) * BT
        offb = jnp.broadcast_to(off_row, (NT, E))
        elo = jnp.sum((offb <= base).astype(jnp.int32), axis=1,
                      keepdims=True) - 1
        ehi = jnp.sum((offb <= base + (BT - 1)).astype(jnp.int32), axis=1,
                      keepdims=True) - 1
        nv = (ehi - elo + 1).astype(jnp.float32)
        tr = lax.broadcasted_iota(jnp.int32, (NT, NT), 0)
        tc = lax.broadcasted_iota(jnp.int32, (NT, NT), 1)
        lt_t = (tc < tr).astype(jnp.float32)
        vstart = lt_t @ nv
        vtotal = jnp.sum(nv)

        vi = lax.broadcasted_iota(jnp.int32, (VPAD, 1), 0)
        vstart_row = lax.dot_general(
            nv, lt_t, (((0,), (1,)), ((), ())))
        vsb = jnp.broadcast_to(vstart_row, (VPAD, NT))
        vif = vi.astype(jnp.float32)
        tv = jnp.sum((vsb <= vif).astype(jnp.int32), axis=1,
                     keepdims=True) - 1
        tiota = lax.broadcasted_iota(jnp.int32, (VPAD, NT), 1)
        oh_tv = (tv == tiota).astype(jnp.float32)
        elo_v = oh_tv @ elo.astype(jnp.float32)
        ehi_v = oh_tv @ ehi.astype(jnp.float32)
        vstart_v = oh_tv @ vstart
        ev = elo_v + vif - vstart_v
        ev = jnp.minimum(ev, ehi_v)
        eviota = lax.broadcasted_iota(jnp.int32, (VPAD, E), 1)
        oh_ev = (ev.astype(jnp.int32) == eviota).astype(jnp.float32)
        offv = jnp.broadcast_to(off_row, (VPAD, E))
        cntv = jnp.broadcast_to(cnt_row, (VPAD, E))
        seg_start = jnp.sum(oh_ev * offv, axis=1, keepdims=True)
        seg_cnt = jnp.sum(oh_ev * cntv, axis=1, keepdims=True)
        tbase = tv.astype(jnp.float32) * BT
        valid = vif < vtotal
        rs = jnp.clip(seg_start - tbase, 0.0, float(BT))
        re = jnp.clip(seg_start + seg_cnt - tbase, 0.0, float(BT))
        rs = jnp.where(valid, rs, 0.0)
        re = jnp.where(valid, re, 0.0)

        meta = jnp.concatenate(
            [tv, ev.astype(jnp.int32), rs.astype(jnp.int32),
             re.astype(jnp.int32)], axis=1)
        meta_ref[...] = meta


def _router_shared(flat, gate_w):
    return pl.pallas_call(
        _router_body,
        grid=(NT,),
        in_specs=[
            pl.BlockSpec((BT, H), lambda t: (t, 0)),
            pl.BlockSpec((H, E), lambda t: (0, 0)),
        ],
        out_specs=[
            pl.BlockSpec((BT, SCW), lambda t: (t, 0)),
            pl.BlockSpec((N, 1), lambda t: (0, 0)),
            pl.BlockSpec((VPAD, 4), lambda t: (0, 0)),
        ],
        out_shape=[
            jax.ShapeDtypeStruct((N, SCW), jnp.float32),
            jax.ShapeDtypeStruct((N, 1), jnp.int32),
            jax.ShapeDtypeStruct((VPAD, 4), jnp.int32),
        ],
        scratch_shapes=[
            pltpu.VMEM((N, E), jnp.float32),
            pltpu.VMEM((N, 1), jnp.float32),
            pltpu.VMEM((1, E), jnp.float32),
        ],
    )(flat, gate_w)


def _grouped_body(meta_ref, xs_ref, sc_ref, wg_ref, wu_ref, wd_ref,
                  sg_ref, su_ref, sd_ref, out_ref):
    v = pl.program_id(0)
    vt = meta_ref[v, 0]
    vt_prev = meta_ref[jnp.maximum(v - 1, 0), 0]
    first = jnp.logical_or(v == 0, vt != vt_prev)
    rs = meta_ref[v, 2]
    re = meta_ref[v, 3]

    x = xs_ref[...]
    s = sc_ref[...][:, 0:1]
    xw = x * s
    ri = lax.broadcasted_iota(jnp.int32, (BT, 1), 0)
    msk = jnp.logical_and(ri >= rs, ri < re)
    xm = jnp.where(msk, xw, 0.0)
    g = xm @ wg_ref[0]
    u = xm @ wu_ref[0]
    y = (_silu(g) * u) @ wd_ref[0]

    @pl.when(first)
    def _():
        sh = _silu(x @ sg_ref[...]) * (x @ su_ref[...])
        out_ref[...] = sh @ sd_ref[...] + y

    @pl.when(jnp.logical_not(first))
    def _():
        out_ref[...] = out_ref[...] + y


def _grouped(meta, xs, scs, wg, wu, wd, sg, su, sd):
    grid_spec = pltpu.PrefetchScalarGridSpec(
        num_scalar_prefetch=1,
        grid=(VMAX,),
        in_specs=[
            pl.BlockSpec((BT, H), lambda v, m: (m[v, 0], 0)),
            pl.BlockSpec((BT, SCW), lambda v, m: (m[v, 0], 0)),
            pl.BlockSpec((1, H, I), lambda v, m: (m[v, 1], 0, 0)),
            pl.BlockSpec((1, H, I), lambda v, m: (m[v, 1], 0, 0)),
            pl.BlockSpec((1, I, H), lambda v, m: (m[v, 1], 0, 0)),
            pl.BlockSpec((H, I), lambda v, m: (0, 0)),
            pl.BlockSpec((H, I), lambda v, m: (0, 0)),
            pl.BlockSpec((I, H), lambda v, m: (0, 0)),
        ],
        out_specs=pl.BlockSpec((BT, H), lambda v, m: (m[v, 0], 0)),
    )
    return pl.pallas_call(
        _grouped_body,
        grid_spec=grid_spec,
        out_shape=jax.ShapeDtypeStruct((N, H), jnp.float32),
    )(meta, xs, scs, wg, wu, wd, sg, su, sd)


def _sc_mesh():
    return plsc.VectorSubcoreMesh(core_axis_name="c", subcore_axis_name="s")


_NW = 32
_RPW = N // _NW


def _scatter_rows(src, scores, pos):
    chunk = 64

    @functools.partial(
        pl.kernel,
        out_type=(
            jax.ShapeDtypeStruct((N, H), jnp.float32),
            jax.ShapeDtypeStruct((N, SCW), jnp.float32),
        ),
        mesh=_sc_mesh(),
        scratch_types=[
            pltpu.VMEM((chunk,), jnp.int32),
            pltpu.VMEM((chunk, H), jnp.float32),
            pltpu.VMEM((chunk, SCW), jnp.float32),
            pltpu.SemaphoreType.DMA,
        ],
    )
    def k(src_hbm, sc_hbm, pos_hbm, outx_hbm, outs_hbm,
          idx_v, rows_v, srow_v, sem):
        wid = lax.axis_index("s") * 2 + lax.axis_index("c")
        base = wid * _RPW
        for ch in range(_RPW // chunk):
            off = base + ch * chunk
            pltpu.sync_copy(pos_hbm.at[pl.ds(off, chunk)], idx_v)
            pltpu.sync_copy(src_hbm.at[pl.ds(off, chunk)], rows_v)
            pltpu.sync_copy(sc_hbm.at[pl.ds(off, chunk)], srow_v)
            cx = pltpu.async_copy(rows_v, outx_hbm.at[idx_v], sem)
            cs = pltpu.async_copy(srow_v, outs_hbm.at[idx_v], sem)
            cx.wait()
            cs.wait()

    return k(src, scores, pos)


def _gather_rows(ys, pos):
    chunk = 64

    @functools.partial(
        pl.kernel,
        out_type=jax.ShapeDtypeStruct((N, H), jnp.float32),
        mesh=_sc_mesh(),
        scratch_types=[
            pltpu.VMEM((chunk,), jnp.int32),
            pltpu.VMEM((chunk, H), jnp.float32),
            pltpu.SemaphoreType.DMA,
        ],
    )
    def k(ys_hbm, pos_hbm, out_hbm, idx_v, rows_v, sem):
        wid = lax.axis_index("s") * 2 + lax.axis_index("c")
        base = wid * _RPW
        for ch in range(_RPW // chunk):
            off = base + ch * chunk
            pltpu.sync_copy(pos_hbm.at[pl.ds(off, chunk)], idx_v)
            pltpu.async_copy(ys_hbm.at[idx_v], rows_v, sem).wait()
            pltpu.sync_copy(rows_v, out_hbm.at[pl.ds(off, chunk)])

    return k(ys, pos)


def kernel(hidden_states, gate_w, shared_gate_w, shared_up_w, shared_down_w,
           routed_gate_w, routed_up_w, routed_down_w):
    flat = hidden_states.reshape(N, H)
    scoreb, pos2d, meta = _router_shared(flat, gate_w)
    pos = pos2d.reshape(N)
    xs, scs = _scatter_rows(flat, scoreb, pos)
    ys = _grouped(meta, xs, scs, routed_gate_w, routed_up_w, routed_down_w,
                  shared_gate_w, shared_up_w, shared_down_w)
    out = _gather_rows(ys, pos)
    return out.reshape(B, S, H)
